# Initial kernel scaffold; baseline (speedup 1.0000x reference)
import jax
import jax.numpy as jnp
from jax.experimental import pallas as pl


def kernel(x, segment_ids, table):
    raise NotImplementedError("write your pallas kernel here")


# speedup vs baseline: 1.1491x; 1.1491x over previous
import functools

import jax
import jax.numpy as jnp
from jax import lax
from jax.experimental import pallas as pl
from jax.experimental.pallas import tpu as pltpu
from jax.experimental.pallas import tpu_sc as plsc

BATCH = 4
SEQ_LEN = 8192
EMBED_DIM = 1024
NUM_SEGMENTS = 16
LANES = 16

N_TOKENS = BATCH * SEQ_LEN
NUM_CORES = 2
NUM_SUBCORES = 16
NUM_WORKERS = NUM_CORES * NUM_SUBCORES
TOKENS_PER_WORKER = N_TOKENS // NUM_WORKERS
CHUNK = 32
NUM_CHUNKS = TOKENS_PER_WORKER // CHUNK


@jax.jit
def _seg_encode(x2d, seg, table):
    mesh = plsc.VectorSubcoreMesh(core_axis_name="c", subcore_axis_name="s")

    @functools.partial(
        pl.kernel,
        mesh=mesh,
        out_type=jax.ShapeDtypeStruct((N_TOKENS, EMBED_DIM), jnp.float32),
        scratch_types=[
            pltpu.VMEM((NUM_SEGMENTS, EMBED_DIM), jnp.float32),
            pltpu.VMEM((CHUNK,), jnp.int32),
            pltpu.VMEM((CHUNK, EMBED_DIM), jnp.float32),
        ],
    )
    def body(x_hbm, seg_hbm, tab_hbm, out_hbm, tab_v, idx_v, xbuf):
        cid = lax.axis_index("c")
        sid = lax.axis_index("s")
        wid = sid * NUM_CORES + cid
        base = wid * TOKENS_PER_WORKER

        pltpu.sync_copy(tab_hbm, tab_v)

        def step(ci, carry):
            off = base + ci * CHUNK
            pltpu.sync_copy(seg_hbm.at[pl.ds(off, CHUNK)], idx_v)
            pltpu.sync_copy(x_hbm.at[pl.ds(off, CHUNK)], xbuf)

            def grp(g, carry2):
                segs = idx_v[pl.ds(g * LANES, LANES)]
                for t16 in range(LANES):
                    s = segs[t16]
                    t = g * LANES + t16

                    @plsc.parallel_loop(0, EMBED_DIM, LANES, unroll=8)
                    def pos(j):
                        plsc.addupdate(
                            xbuf.at[t, pl.ds(j, LANES)],
                            tab_v[s, pl.ds(j, LANES)],
                        )

                return carry2

            lax.fori_loop(0, CHUNK // LANES, grp, 0)

            pltpu.sync_copy(xbuf, out_hbm.at[pl.ds(off, CHUNK)])
            return carry

        lax.fori_loop(0, NUM_CHUNKS, step, 0)

    return body(x2d, seg, table)


def kernel(x, segment_ids, table):
    x2d = x.reshape(N_TOKENS, EMBED_DIM)
    seg = segment_ids.reshape(N_TOKENS).astype(jnp.int32)
    out = _seg_encode(x2d, seg, table)
    return out.reshape(BATCH, SEQ_LEN, EMBED_DIM)


# speedup vs baseline: 1.5846x; 1.3790x over previous
import functools

import jax
import jax.numpy as jnp
from jax import lax
from jax.experimental import pallas as pl
from jax.experimental.pallas import tpu as pltpu
from jax.experimental.pallas import tpu_sc as plsc

BATCH = 4
SEQ_LEN = 8192
EMBED_DIM = 1024
NUM_SEGMENTS = 16
LANES = 16

N_TOKENS = BATCH * SEQ_LEN
NUM_CORES = 2
NUM_SUBCORES = 16
NUM_WORKERS = NUM_CORES * NUM_SUBCORES
TOKENS_PER_WORKER = N_TOKENS // NUM_WORKERS
CHUNK = 32
NUM_CHUNKS = TOKENS_PER_WORKER // CHUNK


@jax.jit
def _seg_encode(x2d, seg, table):
    mesh = plsc.VectorSubcoreMesh(core_axis_name="c", subcore_axis_name="s")

    @functools.partial(
        pl.kernel,
        mesh=mesh,
        out_type=jax.ShapeDtypeStruct((N_TOKENS, EMBED_DIM), jnp.float32),
        scratch_types=[
            pltpu.VMEM((NUM_SEGMENTS, EMBED_DIM), jnp.float32),
            pltpu.VMEM((TOKENS_PER_WORKER,), jnp.int32),
            pltpu.VMEM((CHUNK, EMBED_DIM), jnp.float32),
            pltpu.VMEM((CHUNK, EMBED_DIM), jnp.float32),
            pltpu.SemaphoreType.DMA,
            pltpu.SemaphoreType.DMA,
            pltpu.SemaphoreType.DMA,
            pltpu.SemaphoreType.DMA,
        ],
    )
    def body(x_hbm, seg_hbm, tab_hbm, out_hbm,
             tab_v, idx_all, xb0, xb1, isem0, isem1, osem0, osem1):
        cid = lax.axis_index("c")
        sid = lax.axis_index("s")
        wid = sid * NUM_CORES + cid
        base = wid * TOKENS_PER_WORKER

        xbufs = (xb0, xb1)
        isems = (isem0, isem1)
        osems = (osem0, osem1)

        pltpu.sync_copy(tab_hbm, tab_v)
        pltpu.sync_copy(seg_hbm.at[pl.ds(base, TOKENS_PER_WORKER)], idx_all)

        def issue_load(b, c):
            pltpu.async_copy(
                x_hbm.at[pl.ds(base + c * CHUNK, CHUNK)], xbufs[b], isems[b])

        def wait_load(b):
            pltpu.make_async_copy(
                x_hbm.at[pl.ds(0, CHUNK)], xbufs[b], isems[b]).wait()

        def issue_store(b, c):
            pltpu.async_copy(
                xbufs[b], out_hbm.at[pl.ds(base + c * CHUNK, CHUNK)], osems[b])

        def wait_store(b):
            pltpu.make_async_copy(
                xbufs[b], out_hbm.at[pl.ds(0, CHUNK)], osems[b]).wait()

        def compute(b, c):
            xbuf = xbufs[b]
            for g in range(CHUNK // LANES):
                segs = idx_all[pl.ds(c * CHUNK + g * LANES, LANES)]
                for t16 in range(LANES):
                    s = segs[t16]
                    t = g * LANES + t16

                    @plsc.parallel_loop(0, EMBED_DIM, LANES, unroll=8)
                    def pos(j):
                        plsc.addupdate(
                            xbuf.at[t, pl.ds(j, LANES)],
                            tab_v[s, pl.ds(j, LANES)],
                        )

        issue_load(0, 0)

        def pair(i, carry):
            ci = 2 * i

            @pl.when(ci > 0)
            def _():
                wait_store(1)

            @pl.when(ci + 1 < NUM_CHUNKS)
            def _():
                issue_load(1, ci + 1)

            wait_load(0)
            compute(0, ci)
            issue_store(0, ci)

            @pl.when(ci + 2 < NUM_CHUNKS)
            def _():
                wait_store(0)
                issue_load(0, ci + 2)

            wait_load(1)
            compute(1, ci + 1)
            issue_store(1, ci + 1)
            return carry

        lax.fori_loop(0, NUM_CHUNKS // 2, pair, 0)
        wait_store(0)
        wait_store(1)

    return body(x2d, seg, table)


def kernel(x, segment_ids, table):
    x2d = x.reshape(N_TOKENS, EMBED_DIM)
    seg = segment_ids.reshape(N_TOKENS).astype(jnp.int32)
    out = _seg_encode(x2d, seg, table)
    return out.reshape(BATCH, SEQ_LEN, EMBED_DIM)


# speedup vs baseline: 1.8676x; 1.1786x over previous
import functools

import jax
import jax.numpy as jnp
from jax import lax
from jax.experimental import pallas as pl
from jax.experimental.pallas import tpu as pltpu
from jax.experimental.pallas import tpu_sc as plsc

BATCH = 4
SEQ_LEN = 8192
EMBED_DIM = 1024
NUM_SEGMENTS = 16
LANES = 16

N_TOKENS = BATCH * SEQ_LEN
NUM_CORES = 2
NUM_SUBCORES = 16
NUM_WORKERS = NUM_CORES * NUM_SUBCORES
TOKENS_PER_WORKER = N_TOKENS // NUM_WORKERS
CHUNK = 16
NUM_CHUNKS = TOKENS_PER_WORKER // CHUNK
NBUF = 4
LEAD = 2


@jax.jit
def _seg_encode(x2d, seg, table):
    mesh = plsc.VectorSubcoreMesh(core_axis_name="c", subcore_axis_name="s")

    @functools.partial(
        pl.kernel,
        mesh=mesh,
        out_type=jax.ShapeDtypeStruct((N_TOKENS, EMBED_DIM), jnp.float32),
        scratch_types=[
            pltpu.VMEM((NUM_SEGMENTS, EMBED_DIM), jnp.float32),
            pltpu.VMEM((TOKENS_PER_WORKER,), jnp.int32),
        ]
        + [pltpu.VMEM((CHUNK, EMBED_DIM), jnp.float32)] * NBUF
        + [pltpu.SemaphoreType.DMA] * (2 * NBUF),
    )
    def body(x_hbm, seg_hbm, tab_hbm, out_hbm, tab_v, idx_all, *bufs_sems):
        xbufs = bufs_sems[:NBUF]
        isems = bufs_sems[NBUF:2 * NBUF]
        osems = bufs_sems[2 * NBUF:3 * NBUF]
        cid = lax.axis_index("c")
        sid = lax.axis_index("s")
        wid = sid * NUM_CORES + cid
        base = wid * TOKENS_PER_WORKER

        pltpu.sync_copy(tab_hbm, tab_v)
        pltpu.sync_copy(seg_hbm.at[pl.ds(base, TOKENS_PER_WORKER)], idx_all)

        def issue_load(b, c):
            pltpu.async_copy(
                x_hbm.at[pl.ds(base + c * CHUNK, CHUNK)], xbufs[b], isems[b])

        def wait_load(b):
            pltpu.make_async_copy(
                x_hbm.at[pl.ds(0, CHUNK)], xbufs[b], isems[b]).wait()

        def issue_store(b, c):
            pltpu.async_copy(
                xbufs[b], out_hbm.at[pl.ds(base + c * CHUNK, CHUNK)], osems[b])

        def wait_store(b):
            pltpu.make_async_copy(
                xbufs[b], out_hbm.at[pl.ds(0, CHUNK)], osems[b]).wait()

        def compute(b, c):
            xbuf = xbufs[b]
            for g in range(CHUNK // LANES):
                segs = idx_all[pl.ds(c * CHUNK + g * LANES, LANES)]
                for t16 in range(LANES):
                    s = segs[t16]
                    t = g * LANES + t16

                    @plsc.parallel_loop(0, EMBED_DIM, LANES, unroll=8)
                    def pos(j):
                        plsc.addupdate(
                            xbuf.at[t, pl.ds(j, LANES)],
                            tab_v[s, pl.ds(j, LANES)],
                        )

        for b in range(LEAD):
            issue_load(b, b)

        def ring(i, carry):
            c0 = NBUF * i
            for b in range(NBUF):
                c = c0 + b
                wait_load(b)
                compute(b, c)
                issue_store(b, c)
                nb = (b + LEAD) % NBUF

                @pl.when(c + LEAD < NUM_CHUNKS)
                def _():
                    @pl.when(c + LEAD >= NBUF)
                    def _():
                        wait_store(nb)

                    issue_load(nb, c + LEAD)

            return carry

        lax.fori_loop(0, NUM_CHUNKS // NBUF, ring, 0)
        for b in range(NBUF):
            wait_store((NUM_CHUNKS - NBUF + b) % NBUF)

    return body(x2d, seg, table)


def kernel(x, segment_ids, table):
    x2d = x.reshape(N_TOKENS, EMBED_DIM)
    seg = segment_ids.reshape(N_TOKENS).astype(jnp.int32)
    out = _seg_encode(x2d, seg, table)
    return out.reshape(BATCH, SEQ_LEN, EMBED_DIM)
